# baseline (speedup 1.0000x reference)
import functools

import jax
import jax.numpy as jnp
from jax import lax
from jax.experimental import pallas as pl
from jax.experimental.pallas import tpu as pltpu
from jax.experimental.pallas import tpu_sc as plsc

_NUM_CORES = 2
_NUM_SUBCORES = 16
_NUM_WORKERS = _NUM_CORES * _NUM_SUBCORES
_CHUNK_ROWS = 16


@functools.partial(jax.jit, static_argnums=(1, 2))
def _position_embedding(table, seq_len, dim):
    rows_per_worker = seq_len // _NUM_WORKERS
    n_chunks = rows_per_worker // _CHUNK_ROWS
    n_pairs = n_chunks // 2
    mesh = plsc.VectorSubcoreMesh(
        core_axis_name="c", subcore_axis_name="s", num_cores=_NUM_CORES
    )

    @functools.partial(
        pl.kernel,
        out_type=jax.ShapeDtypeStruct((seq_len, dim), table.dtype),
        mesh=mesh,
        scratch_types=[
            pltpu.VMEM((2, _CHUNK_ROWS, dim), table.dtype),
            pltpu.SemaphoreType.DMA((2,)),
            pltpu.SemaphoreType.DMA((2,)),
        ],
    )
    def copy_kernel(table_hbm, out_hbm, buf, in_sems, out_sems):
        wid = lax.axis_index("s") * _NUM_CORES + lax.axis_index("c")
        base = wid * rows_per_worker

        def chunk_in(c):
            return table_hbm.at[pl.ds(base + c * _CHUNK_ROWS, _CHUNK_ROWS)]

        def chunk_out(c):
            return out_hbm.at[pl.ds(base + c * _CHUNK_ROWS, _CHUNK_ROWS)]

        def wait_in(b):
            pltpu.make_async_copy(chunk_in(0), buf.at[b], in_sems.at[b]).wait()

        def wait_out(b):
            pltpu.make_async_copy(buf.at[b], chunk_out(0), out_sems.at[b]).wait()

        pltpu.async_copy(chunk_in(0), buf.at[0], in_sems.at[0])

        def pair(k, _):
            c0 = 2 * k

            @pl.when(k >= 1)
            def _():
                wait_out(1)

            pltpu.async_copy(chunk_in(c0 + 1), buf.at[1], in_sems.at[1])
            wait_in(0)
            pltpu.async_copy(buf.at[0], chunk_out(c0), out_sems.at[0])

            @pl.when(k + 1 < n_pairs)
            def _():
                wait_out(0)
                pltpu.async_copy(chunk_in(c0 + 2), buf.at[0], in_sems.at[0])

            wait_in(1)
            pltpu.async_copy(buf.at[1], chunk_out(c0 + 1), out_sems.at[1])
            return 0

        lax.fori_loop(0, n_pairs, pair, 0, unroll=False)
        wait_out(0)
        wait_out(1)

    return copy_kernel(table)


def kernel(inputs, table):
    seq_len = inputs.shape[-1]
    return _position_embedding(table, seq_len, table.shape[1])


# speedup vs baseline: 1.0433x; 1.0433x over previous
import functools

import jax
import jax.numpy as jnp
from jax import lax
from jax.experimental import pallas as pl
from jax.experimental.pallas import tpu as pltpu
from jax.experimental.pallas import tpu_sc as plsc

_NUM_CORES = 2
_NUM_SUBCORES = 16
_NUM_WORKERS = _NUM_CORES * _NUM_SUBCORES
_CHUNK_ROWS = 32
_NBUF = 3


@functools.partial(jax.jit, static_argnums=(1, 2))
def _position_embedding(table, seq_len, dim):
    rows_per_worker = seq_len // _NUM_WORKERS
    n_chunks = rows_per_worker // _CHUNK_ROWS
    mesh = plsc.VectorSubcoreMesh(
        core_axis_name="c", subcore_axis_name="s", num_cores=_NUM_CORES
    )

    @functools.partial(
        pl.kernel,
        out_type=jax.ShapeDtypeStruct((seq_len, dim), table.dtype),
        mesh=mesh,
        scratch_types=[
            pltpu.VMEM((_NBUF, _CHUNK_ROWS, dim), table.dtype),
            pltpu.SemaphoreType.DMA((_NBUF,)),
            pltpu.SemaphoreType.DMA((_NBUF,)),
        ],
    )
    def copy_kernel(table_hbm, out_hbm, buf, in_sems, out_sems):
        wid = lax.axis_index("s") * _NUM_CORES + lax.axis_index("c")
        base = wid * rows_per_worker

        def fire_in(c):
            b = c % _NBUF
            return pltpu.async_copy(
                table_hbm.at[pl.ds(base + c * _CHUNK_ROWS, _CHUNK_ROWS)],
                buf.at[b],
                in_sems.at[b],
            )

        def fire_out(c):
            b = c % _NBUF
            return pltpu.async_copy(
                buf.at[b],
                out_hbm.at[pl.ds(base + c * _CHUNK_ROWS, _CHUNK_ROWS)],
                out_sems.at[b],
            )

        in_dma, out_dma = {}, {}
        for c in range(min(_NBUF, n_chunks)):
            in_dma[c] = fire_in(c)
        fired = min(_NBUF, n_chunks)
        unwaited = []
        for c in range(n_chunks):
            in_dma[c].wait()
            out_dma[c] = fire_out(c)
            unwaited.append(c)
            if fired < n_chunks:
                out_dma[fired - _NBUF].wait()
                unwaited.remove(fired - _NBUF)
                in_dma[fired] = fire_in(fired)
                fired += 1
        for c in unwaited:
            out_dma[c].wait()

    return copy_kernel(table)


def kernel(inputs, table):
    seq_len = inputs.shape[-1]
    return _position_embedding(table, seq_len, table.shape[1])
